# baseline (speedup 1.0000x reference)
import functools

import jax
import jax.numpy as jnp
from jax import lax
from jax.experimental import pallas as pl
from jax.experimental.pallas import tpu as pltpu
from jax.experimental.pallas import tpu_sc as plsc

N_NODES_C = 10000
D_C = 128
E_C = 320000
B_C = 256
K_C = 200
NBLK = 1024
NPAD = 10240
ALPHA_C = 0.5

NC = 2
NS = 16
NW = NC * NS
EWP = 10240
EROWS = EWP // 128
SL = NPAD // NS

def _sc_mesh():
    return plsc.VectorSubcoreMesh(core_axis_name="c", subcore_axis_name="s",
                                  num_cores=NC, num_subcores=NS)


def _deg_body(dst3_hbm, out_hbm, dst_v, ones_v, slice_v, deg_sh, sem):
    c = lax.axis_index("c")
    s = lax.axis_index("s")
    wid = s * NC + c

    pltpu.sync_copy(dst3_hbm.at[wid], dst_v)

    def fill(i, carry):
        ones_v[pl.ds(pl.multiple_of(i * 16, 16), 16)] = jnp.full((16,), 1.0,
                                                                 jnp.float32)
        return carry

    lax.fori_loop(0, EWP // 16, fill, 0)

    def zero(i, carry):
        slice_v[pl.ds(pl.multiple_of(i * 16, 16), 16)] = jnp.zeros(
            (16,), jnp.float32)
        return carry

    lax.fori_loop(0, SL // 16, zero, 0)
    pltpu.sync_copy(slice_v, deg_sh.at[pl.ds(s * SL, SL)])
    plsc.subcore_barrier()

    def scat(j, carry):
        pltpu.sync_copy(ones_v.at[pl.ds(pl.multiple_of(j * 128, 128), 128)],
                        deg_sh.at[dst_v.at[j]], add=True)
        return carry

    lax.fori_loop(0, EROWS, scat, 0)
    plsc.subcore_barrier()

    pltpu.sync_copy(deg_sh.at[pl.ds(s * SL, SL)], slice_v)
    pltpu.sync_copy(slice_v, out_hbm.at[c, pl.ds(s * SL, SL)])


def _deg_call(dst3):
    return pl.kernel(
        _deg_body,
        out_type=jax.ShapeDtypeStruct((NC, NPAD), jnp.float32),
        mesh=_sc_mesh(),
        scratch_types=[
            pltpu.VMEM((EROWS, 128), jnp.int32),
            pltpu.VMEM((EWP,), jnp.float32),
            pltpu.VMEM((SL,), jnp.float32),
            pltpu.VMEM_SHARED((NPAD,), jnp.float32),
            pltpu.SemaphoreType.DMA,
        ],
    )(dst3)


def _edge_body(dst3_hbm, dstf_hbm, srcf_hbm, dinv_hbm, hv_hbm,
               rs_hbm, hm_hbm,
               dst_v, dstf_v, srcf_v, dinv_v, hv_v, ev_v, hmv_v, slice_v,
               rs_sh, hm_sh, sem):
    c = lax.axis_index("c")
    s = lax.axis_index("s")
    wid = s * NC + c

    pltpu.sync_copy(dst3_hbm.at[wid], dst_v)
    pltpu.sync_copy(dstf_hbm.at[wid], dstf_v)
    pltpu.sync_copy(srcf_hbm.at[wid], srcf_v)
    pltpu.sync_copy(dinv_hbm, dinv_v)
    pltpu.sync_copy(hv_hbm, hv_v)

    def zero(i, carry):
        slice_v[pl.ds(pl.multiple_of(i * 16, 16), 16)] = jnp.zeros(
            (16,), jnp.float32)
        return carry

    lax.fori_loop(0, SL // 16, zero, 0)
    pltpu.sync_copy(slice_v, rs_sh.at[pl.ds(s * SL, SL)])
    pltpu.sync_copy(slice_v, hm_sh.at[pl.ds(s * SL, SL)])
    plsc.subcore_barrier()

    def gath(i, carry):
        off = pl.ds(pl.multiple_of(i * 16, 16), 16)
        id_d = dstf_v[off]
        id_s = srcf_v[off]
        dv = plsc.load_gather(dinv_v, [id_d])
        sv = plsc.load_gather(dinv_v, [id_s])
        ev_v[off] = dv * sv
        hmv_v[off] = plsc.load_gather(hv_v, [id_s])
        return carry

    lax.fori_loop(0, EWP // 16, gath, 0)

    def scat(j, carry):
        row = pl.ds(pl.multiple_of(j * 128, 128), 128)
        pltpu.sync_copy(ev_v.at[row], rs_sh.at[dst_v.at[j]], add=True)
        pltpu.sync_copy(hmv_v.at[row], hm_sh.at[dst_v.at[j]], add=True)
        return carry

    lax.fori_loop(0, EROWS, scat, 0)
    plsc.subcore_barrier()

    sl = pl.ds(s * SL, SL)
    pltpu.sync_copy(rs_sh.at[sl], slice_v)
    pltpu.sync_copy(slice_v, rs_hbm.at[c, sl])
    pltpu.sync_copy(hm_sh.at[sl], slice_v)
    pltpu.sync_copy(slice_v, hm_hbm.at[c, sl])


def _edge_call(dst3, dstf, srcf, dinv, hv):
    return pl.kernel(
        _edge_body,
        out_type=(jax.ShapeDtypeStruct((NC, NPAD), jnp.float32),
                  jax.ShapeDtypeStruct((NC, NPAD), jnp.float32)),
        mesh=_sc_mesh(),
        compiler_params=pltpu.CompilerParams(needs_layout_passes=False),
        scratch_types=[
            pltpu.VMEM((EROWS, 128), jnp.int32),
            pltpu.VMEM((EWP,), jnp.int32),
            pltpu.VMEM((EWP,), jnp.int32),
            pltpu.VMEM((NPAD,), jnp.float32),
            pltpu.VMEM((NPAD,), jnp.float32),
            pltpu.VMEM((EWP,), jnp.float32),
            pltpu.VMEM((EWP,), jnp.float32),
            pltpu.VMEM((SL,), jnp.float32),
            pltpu.VMEM_SHARED((NPAD,), jnp.float32),
            pltpu.VMEM_SHARED((NPAD,), jnp.float32),
            pltpu.SemaphoreType.DMA,
        ],
    )(dst3, dstf, srcf, dinv, hv)


def _hv_body(xblk_ref, wmat_ref, out_ref):
    out_ref[...] = jax.lax.dot_general(
        wmat_ref[...], xblk_ref[...], (((1,), (1,)), ((), ())),
        preferred_element_type=jnp.float32)


def _hv_call(xpad, wmat):
    return pl.pallas_call(
        _hv_body,
        grid=(NPAD // NBLK,),
        in_specs=[
            pl.BlockSpec((NBLK, D_C), lambda j: (j, 0)),
            pl.BlockSpec((8, D_C), lambda j: (0, 0)),
        ],
        out_specs=pl.BlockSpec((8, NBLK), lambda j: (0, j)),
        out_shape=jax.ShapeDtypeStruct((8, NPAD), jnp.float32),
    )(xpad, wmat)


def _dense_body(xb_ref, xblk_ref, wr_ref, wu_ref, wth_ref, nimp_ref,
                layer_ref, lr_root_ref, ni_root_ref, p_ref, th_ref):
    j = pl.program_id(0)
    xb = xb_ref[...]
    a = xb * wr_ref[...]
    an = jnp.maximum(jnp.sqrt(jnp.sum(a * a, axis=1, keepdims=True)), 1e-6)
    bb = xb * wu_ref[...]
    bn_r = jnp.maximum(jnp.sqrt(jnp.sum(bb * bb, axis=1, keepdims=True)), 1e-6)
    ego_root = jnp.sum(a * bb, axis=1, keepdims=True) / (an * bn_r)
    p_root = (ALPHA_C * ego_root
              + (1.0 - ALPHA_C) * lr_root_ref[...]) * ni_root_ref[...]

    b = xblk_ref[...] * wu_ref[...]
    bn = jnp.maximum(jnp.sqrt(jnp.sum(b * b, axis=1, keepdims=True)), 1e-6)
    num = jax.lax.dot_general(a, b, (((1,), (1,)), ((), ())),
                              preferred_element_type=jnp.float32)
    ego = num / (an * bn.T)
    p = (ALPHA_C * ego + (1.0 - ALPHA_C) * layer_ref[...]) * nimp_ref[...]
    p = p / (p_root + 1e-7) + 1.0
    p = jnp.clip(p, 0.01, 1.0)
    col = j * NBLK + jax.lax.broadcasted_iota(jnp.int32, (1, NBLK), 1)
    p_ref[...] = jnp.where(col < N_NODES_C, p, -1.0)
    th_ref[...] = jax.lax.dot_general(xb, wth_ref[...], (((1,), (0,)), ((), ())),
                                      preferred_element_type=jnp.float32)


def _dense_call(xb, xpad, w_r, w_u, w_th, nimp_pad, layer_pad, lr_root,
                ni_root):
    grid = (NPAD // NBLK,)
    return pl.pallas_call(
        _dense_body,
        grid=grid,
        in_specs=[
            pl.BlockSpec((B_C, D_C), lambda j: (0, 0)),
            pl.BlockSpec((NBLK, D_C), lambda j: (j, 0)),
            pl.BlockSpec((1, D_C), lambda j: (0, 0)),
            pl.BlockSpec((1, D_C), lambda j: (0, 0)),
            pl.BlockSpec((D_C, 1), lambda j: (0, 0)),
            pl.BlockSpec((1, NBLK), lambda j: (0, j)),
            pl.BlockSpec((1, NBLK), lambda j: (0, j)),
            pl.BlockSpec((B_C, 1), lambda j: (0, 0)),
            pl.BlockSpec((B_C, 1), lambda j: (0, 0)),
        ],
        out_specs=[
            pl.BlockSpec((B_C, NBLK), lambda j: (0, j)),
            pl.BlockSpec((B_C, 1), lambda j: (0, 0)),
        ],
        out_shape=[
            jax.ShapeDtypeStruct((B_C, NPAD), jnp.float32),
            jax.ShapeDtypeStruct((B_C, 1), jnp.float32),
        ],
    )(xb, xpad, w_r, w_u, w_th, nimp_pad, layer_pad, lr_root, ni_root)


def kernel(x, edge_index, batch_nodes, w_ego_root, w_ego_u, w_layer_v,
           w_layer_u, w_threshold):
    N = x.shape[0]
    src = edge_index[0]
    dst = edge_index[1]

    epad = NW * EWP - E_C
    pad_idx = jnp.full((epad,), N_NODES_C, jnp.int32)
    dstp = jnp.concatenate([dst, pad_idx])
    srcp = jnp.concatenate([src, pad_idx])
    dst3 = dstp.reshape(NW, EROWS, 128)
    dstf = dstp.reshape(NW, EWP)
    srcf = srcp.reshape(NW, EWP)
    xpad = jnp.pad(x, ((0, NPAD - N), (0, 0)))

    wmat = jnp.zeros((8, D_C), jnp.float32)
    wmat = wmat.at[0].set(w_layer_v.reshape(-1)).at[1].set(w_layer_u.reshape(-1))
    hv_uv = _hv_call(xpad, wmat)
    h_v = hv_uv[0]
    u_v = hv_uv[1]

    deg_part = _deg_call(dst3)
    deg_inv = 1.0 / (deg_part[0] + deg_part[1] + 1.0)

    rs_part, hm_part = _edge_call(dst3, dstf, srcf, deg_inv, h_v)

    rowsum = rs_part[0] + rs_part[1] + deg_inv * deg_inv
    n_imp = jnp.sqrt(rowsum)
    h = jax.nn.relu(hm_part[0] + hm_part[1] + u_v)[:N]
    layer_score = h / jnp.maximum(jnp.linalg.norm(h), 1e-12)

    xb = x[batch_nodes]
    lr_root = layer_score[batch_nodes].reshape(B_C, 1)
    ni_root = n_imp[batch_nodes].reshape(B_C, 1)

    nimp_pad = n_imp.reshape(1, NPAD)
    layer_pad = jnp.pad(layer_score, (0, NPAD - N)).reshape(1, NPAD)

    p_clip, th = _dense_call(xb, xpad, w_ego_root.reshape(1, D_C),
                             w_ego_u.reshape(1, D_C), w_threshold,
                             nimp_pad, layer_pad, lr_root, ni_root)

    vals, idx = jax.lax.top_k(p_clip[:, :N], K_C)
    return vals, idx, th.reshape(-1)


# speedup vs baseline: 1.7281x; 1.7281x over previous
import functools

import jax
import jax.numpy as jnp
from jax import lax
from jax.experimental import pallas as pl
from jax.experimental.pallas import tpu as pltpu
from jax.experimental.pallas import tpu_sc as plsc

N_NODES_C = 10000
D_C = 128
E_C = 320000
B_C = 256
K_C = 200
NBLK = 1024
NPAD = 10240
ALPHA_C = 0.5

NC = 2
NS = 16
NW = NC * NS
EWP = 10240
EROWS = EWP // 128
SL = NPAD // NS

def _sc_mesh():
    return plsc.VectorSubcoreMesh(core_axis_name="c", subcore_axis_name="s",
                                  num_cores=NC, num_subcores=NS)


def _deg_body(dst3_hbm, out_hbm, dst_v, ones_v, slice_v, deg_sh, sem):
    c = lax.axis_index("c")
    s = lax.axis_index("s")
    wid = s * NC + c

    pltpu.sync_copy(dst3_hbm.at[wid], dst_v)

    def fill(i, carry):
        ones_v[pl.ds(pl.multiple_of(i * 16, 16), 16)] = jnp.full((16,), 1.0,
                                                                 jnp.float32)
        return carry

    lax.fori_loop(0, EWP // 16, fill, 0)

    def zero(i, carry):
        slice_v[pl.ds(pl.multiple_of(i * 16, 16), 16)] = jnp.zeros(
            (16,), jnp.float32)
        return carry

    lax.fori_loop(0, SL // 16, zero, 0)
    pltpu.sync_copy(slice_v, deg_sh.at[pl.ds(s * SL, SL)])
    plsc.subcore_barrier()

    def scat(j, carry):
        pltpu.sync_copy(ones_v.at[pl.ds(pl.multiple_of(j * 128, 128), 128)],
                        deg_sh.at[dst_v.at[j]], add=True)
        return carry

    lax.fori_loop(0, EROWS, scat, 0)
    plsc.subcore_barrier()

    pltpu.sync_copy(deg_sh.at[pl.ds(s * SL, SL)], slice_v)
    pltpu.sync_copy(slice_v, out_hbm.at[c, pl.ds(s * SL, SL)])


def _deg_call(dst3):
    return pl.kernel(
        _deg_body,
        out_type=jax.ShapeDtypeStruct((NC, NPAD), jnp.float32),
        mesh=_sc_mesh(),
        scratch_types=[
            pltpu.VMEM((EROWS, 128), jnp.int32),
            pltpu.VMEM((EWP,), jnp.float32),
            pltpu.VMEM((SL,), jnp.float32),
            pltpu.VMEM_SHARED((NPAD,), jnp.float32),
            pltpu.SemaphoreType.DMA,
        ],
    )(dst3)


def _edge_body(dst3_hbm, dstf_hbm, srcf_hbm, dinv_hbm, hv_hbm,
               rs_hbm, hm_hbm,
               dst_v, dstf_v, srcf_v, dinv_v, hv_v, ev_v, hmv_v, slice_v,
               rs_sh, hm_sh, sem):
    c = lax.axis_index("c")
    s = lax.axis_index("s")
    wid = s * NC + c

    pltpu.sync_copy(dst3_hbm.at[wid], dst_v)
    pltpu.sync_copy(dstf_hbm.at[wid], dstf_v)
    pltpu.sync_copy(srcf_hbm.at[wid], srcf_v)
    pltpu.sync_copy(dinv_hbm, dinv_v)
    pltpu.sync_copy(hv_hbm, hv_v)

    def zero(i, carry):
        slice_v[pl.ds(pl.multiple_of(i * 16, 16), 16)] = jnp.zeros(
            (16,), jnp.float32)
        return carry

    lax.fori_loop(0, SL // 16, zero, 0)
    pltpu.sync_copy(slice_v, rs_sh.at[pl.ds(s * SL, SL)])
    pltpu.sync_copy(slice_v, hm_sh.at[pl.ds(s * SL, SL)])
    plsc.subcore_barrier()

    def gath(i, carry):
        off = pl.ds(pl.multiple_of(i * 16, 16), 16)
        id_d = dstf_v[off]
        id_s = srcf_v[off]
        dv = plsc.load_gather(dinv_v, [id_d])
        sv = plsc.load_gather(dinv_v, [id_s])
        ev_v[off] = dv * sv
        hmv_v[off] = plsc.load_gather(hv_v, [id_s])
        return carry

    lax.fori_loop(0, EWP // 16, gath, 0)

    def scat(j, carry):
        row = pl.ds(pl.multiple_of(j * 128, 128), 128)
        pltpu.sync_copy(ev_v.at[row], rs_sh.at[dst_v.at[j]], add=True)
        pltpu.sync_copy(hmv_v.at[row], hm_sh.at[dst_v.at[j]], add=True)
        return carry

    lax.fori_loop(0, EROWS, scat, 0)
    plsc.subcore_barrier()

    sl = pl.ds(s * SL, SL)
    pltpu.sync_copy(rs_sh.at[sl], slice_v)
    pltpu.sync_copy(slice_v, rs_hbm.at[c, sl])
    pltpu.sync_copy(hm_sh.at[sl], slice_v)
    pltpu.sync_copy(slice_v, hm_hbm.at[c, sl])


def _edge_call(dst3, dstf, srcf, dinv, hv):
    return pl.kernel(
        _edge_body,
        out_type=(jax.ShapeDtypeStruct((NC, NPAD), jnp.float32),
                  jax.ShapeDtypeStruct((NC, NPAD), jnp.float32)),
        mesh=_sc_mesh(),
        compiler_params=pltpu.CompilerParams(needs_layout_passes=False),
        scratch_types=[
            pltpu.VMEM((EROWS, 128), jnp.int32),
            pltpu.VMEM((EWP,), jnp.int32),
            pltpu.VMEM((EWP,), jnp.int32),
            pltpu.VMEM((NPAD,), jnp.float32),
            pltpu.VMEM((NPAD,), jnp.float32),
            pltpu.VMEM((EWP,), jnp.float32),
            pltpu.VMEM((EWP,), jnp.float32),
            pltpu.VMEM((SL,), jnp.float32),
            pltpu.VMEM_SHARED((NPAD,), jnp.float32),
            pltpu.VMEM_SHARED((NPAD,), jnp.float32),
            pltpu.SemaphoreType.DMA,
        ],
    )(dst3, dstf, srcf, dinv, hv)


def _hv_body(xblk_ref, wmat_ref, out_ref):
    out_ref[...] = jax.lax.dot_general(
        wmat_ref[...], xblk_ref[...], (((1,), (1,)), ((), ())),
        preferred_element_type=jnp.float32)


def _hv_call(xpad, wmat):
    return pl.pallas_call(
        _hv_body,
        grid=(NPAD // NBLK,),
        in_specs=[
            pl.BlockSpec((NBLK, D_C), lambda j: (j, 0)),
            pl.BlockSpec((8, D_C), lambda j: (0, 0)),
        ],
        out_specs=pl.BlockSpec((8, NBLK), lambda j: (0, j)),
        out_shape=jax.ShapeDtypeStruct((8, NPAD), jnp.float32),
    )(xpad, wmat)


def _dense_body(xb_ref, xblk_ref, wr_ref, wu_ref, wth_ref, nimp_ref,
                layer_ref, lr_root_ref, ni_root_ref, p_ref, th_ref):
    j = pl.program_id(0)
    xb = xb_ref[...]
    a = xb * wr_ref[...]
    an = jnp.maximum(jnp.sqrt(jnp.sum(a * a, axis=1, keepdims=True)), 1e-6)
    bb = xb * wu_ref[...]
    bn_r = jnp.maximum(jnp.sqrt(jnp.sum(bb * bb, axis=1, keepdims=True)), 1e-6)
    ego_root = jnp.sum(a * bb, axis=1, keepdims=True) / (an * bn_r)
    p_root = (ALPHA_C * ego_root
              + (1.0 - ALPHA_C) * lr_root_ref[...]) * ni_root_ref[...]

    b = xblk_ref[...] * wu_ref[...]
    bn = jnp.maximum(jnp.sqrt(jnp.sum(b * b, axis=1, keepdims=True)), 1e-6)
    num = jax.lax.dot_general(a, b, (((1,), (1,)), ((), ())),
                              preferred_element_type=jnp.float32)
    ego = num / (an * bn.T)
    p = (ALPHA_C * ego + (1.0 - ALPHA_C) * layer_ref[...]) * nimp_ref[...]
    p = p / (p_root + 1e-7) + 1.0
    p = jnp.clip(p, 0.01, 1.0)
    col = j * NBLK + jax.lax.broadcasted_iota(jnp.int32, (1, NBLK), 1)
    p_ref[...] = jnp.where(col < N_NODES_C, p, -1.0)
    th_ref[...] = jax.lax.dot_general(xb, wth_ref[...], (((1,), (0,)), ((), ())),
                                      preferred_element_type=jnp.float32)


def _dense_call(xb, xpad, w_r, w_u, w_th, nimp_pad, layer_pad, lr_root,
                ni_root):
    grid = (NPAD // NBLK,)
    return pl.pallas_call(
        _dense_body,
        grid=grid,
        in_specs=[
            pl.BlockSpec((B_C, D_C), lambda j: (0, 0)),
            pl.BlockSpec((NBLK, D_C), lambda j: (j, 0)),
            pl.BlockSpec((1, D_C), lambda j: (0, 0)),
            pl.BlockSpec((1, D_C), lambda j: (0, 0)),
            pl.BlockSpec((D_C, 1), lambda j: (0, 0)),
            pl.BlockSpec((1, NBLK), lambda j: (0, j)),
            pl.BlockSpec((1, NBLK), lambda j: (0, j)),
            pl.BlockSpec((B_C, 1), lambda j: (0, 0)),
            pl.BlockSpec((B_C, 1), lambda j: (0, 0)),
        ],
        out_specs=[
            pl.BlockSpec((B_C, NBLK), lambda j: (0, j)),
            pl.BlockSpec((B_C, 1), lambda j: (0, 0)),
        ],
        out_shape=[
            jax.ShapeDtypeStruct((B_C, NPAD), jnp.float32),
            jax.ShapeDtypeStruct((B_C, 1), jnp.float32),
        ],
    )(xb, xpad, w_r, w_u, w_th, nimp_pad, layer_pad, lr_root, ni_root)


def kernel(x, edge_index, batch_nodes, w_ego_root, w_ego_u, w_layer_v,
           w_layer_u, w_threshold):
    N = x.shape[0]
    src = edge_index[0]
    dst = edge_index[1]

    epad = NW * EWP - E_C
    pad_idx = jnp.full((epad,), N_NODES_C, jnp.int32)
    dstp = jnp.concatenate([dst, pad_idx])
    srcp = jnp.concatenate([src, pad_idx])
    dst3 = dstp.reshape(NW, EROWS, 128)
    dstf = dstp.reshape(NW, EWP)
    srcf = srcp.reshape(NW, EWP)
    xpad = jnp.pad(x, ((0, NPAD - N), (0, 0)))

    wmat = jnp.zeros((8, D_C), jnp.float32)
    wmat = wmat.at[0].set(w_layer_v.reshape(-1)).at[1].set(w_layer_u.reshape(-1))
    hv_uv = _hv_call(xpad, wmat)
    h_v = hv_uv[0]
    u_v = hv_uv[1]

    deg_part = _deg_call(dst3)
    deg_inv = 1.0 / (deg_part[0] + deg_part[1] + 1.0)

    rs_part, hm_part = _edge_call(dst3, dstf, srcf, deg_inv, h_v)

    rowsum = rs_part[0] + rs_part[1] + deg_inv * deg_inv
    n_imp = jnp.sqrt(rowsum)
    h = jax.nn.relu(hm_part[0] + hm_part[1] + u_v)[:N]
    layer_score = h / jnp.maximum(jnp.linalg.norm(h), 1e-12)

    xb = x[batch_nodes]
    lr_root = layer_score[batch_nodes].reshape(B_C, 1)
    ni_root = n_imp[batch_nodes].reshape(B_C, 1)

    nimp_pad = n_imp.reshape(1, NPAD)
    layer_pad = jnp.pad(layer_score, (0, NPAD - N)).reshape(1, NPAD)

    p_clip, th = _dense_call(xb, xpad, w_ego_root.reshape(1, D_C),
                             w_ego_u.reshape(1, D_C), w_threshold,
                             nimp_pad, layer_pad, lr_root, ni_root)

    nseg = NPAD // NBLK
    pseg = p_clip.reshape(B_C * nseg, NBLK)
    sv, si = jax.lax.top_k(pseg, K_C)
    gi = si.reshape(B_C, nseg, K_C) + (jnp.arange(nseg, dtype=jnp.int32)
                                       * NBLK)[None, :, None]
    cand_v = sv.reshape(B_C, nseg * K_C)
    cand_i = gi.reshape(B_C, nseg * K_C)
    vals, pos = jax.lax.top_k(cand_v, K_C)
    idx = jnp.take_along_axis(cand_i, pos, axis=1)
    return vals, idx, th.reshape(-1)
